# baseline (speedup 1.0000x reference)
import functools

import jax
import jax.numpy as jnp
from jax import lax
from jax.experimental import pallas as pl
from jax.experimental.pallas import tpu as pltpu
from jax.experimental.pallas import tpu_sc as plsc

NH = 16
MAXD = 32
S = 2048
W = 2 * MAXD + 1
NP = 128
MPAD = 4096
GR = 8
GPH = 16
HPC = NH // 2


def _build_body(table_ref, sl_ref):
    t = jax.lax.broadcasted_iota(jnp.int32, (1, MPAD), 1)
    idx = jnp.clip(t - S, -MAXD, MAXD) + MAXD
    acc = jnp.full((1, MPAD), table_ref[0, 0, 0], dtype=jnp.float32)
    for k in range(1, W):
        acc = jnp.where(idx == k, table_ref[0, 0, k], acc)
    bm = jnp.broadcast_to(acc, (NP, MPAD))
    sl_ref[0] = pltpu.roll(bm, 0, 1, stride=1, stride_axis=0)


def _build_slab(relative_bias):
    return pl.pallas_call(
        _build_body,
        grid=(NH,),
        in_specs=[
            pl.BlockSpec((1, 1, W), lambda h: (h, 0, 0),
                         memory_space=pltpu.SMEM),
        ],
        out_specs=pl.BlockSpec((1, NP, MPAD), lambda h: (h, 0, 0)),
        out_shape=jax.ShapeDtypeStruct((NH, NP, MPAD), jnp.float32),
    )(relative_bias.reshape(NH, 1, W))


def _sc_materialize(slab):
    info = plsc.get_sparse_core_info()
    nc, ns = info.num_cores, info.num_subcores
    assert nc == 2 and ns == 16
    mesh = plsc.VectorSubcoreMesh(core_axis_name="c", subcore_axis_name="s")

    @functools.partial(
        pl.kernel,
        mesh=mesh,
        out_type=jax.ShapeDtypeStruct((NH, S, S), jnp.float32),
        scratch_types=[
            pltpu.VMEM((3, GR, MPAD), jnp.float32),
            pltpu.SemaphoreType.DMA((3,)),
            pltpu.SemaphoreType.DMA((3,)),
        ],
    )
    def sc_k(sl_hbm, out_hbm, m_v, ssem, gsem):
        sid = lax.axis_index("s")
        cid = lax.axis_index("c")
        rb = pl.multiple_of(GR * sid, GR)

        def stage(hh, sl):
            return pltpu.make_async_copy(
                sl_hbm.at[HPC * cid + hh, pl.ds(rb, GR), :],
                m_v.at[sl], ssem.at[sl])

        def group(hh, m, sl):
            i_g = pl.multiple_of(GR * sid + NP * m, GR)
            v0 = pl.multiple_of(S - NP * m, NP)
            return pltpu.make_async_copy(
                m_v.at[sl, :, pl.ds(v0, S)],
                out_hbm.at[HPC * cid + hh, pl.ds(i_g, GR), :],
                gsem.at[sl],
            )

        stage(0, 0).start()
        stage(1, 1).start()

        def head_step(hh, carry):
            sl = lax.rem(hh, 3)

            @pl.when(hh >= 1)
            def _drain_prev():
                for m in range(GPH):
                    group(hh - 1, m, lax.rem(hh - 1, 3)).wait()

            stage(hh, sl).wait()
            for m in range(GPH):
                group(hh, m, sl).start()

            @pl.when(hh + 2 < HPC)
            def _next():
                stage(hh + 2, lax.rem(hh + 2, 3)).start()

            return carry

        lax.fori_loop(0, HPC, head_step, 0)
        for m in range(GPH):
            group(HPC - 1, m, lax.rem(HPC - 1, 3)).wait()

    return sc_k(slab)


def kernel(seq_len, relative_bias):
    del seq_len
    return _sc_materialize(_build_slab(relative_bias))
